# Initial kernel scaffold; baseline (speedup 1.0000x reference)
import jax
import jax.numpy as jnp
from jax.experimental import pallas as pl


def kernel(x, pos, W_initial, b_initial, W_withpos, b_withpos, W_s1, b_s1, W_s2, b_s2, W_s3, b_s3, W_s4, b_s4, W_sbconv, b_sbconv, W_values, b_values, W_select, b_select, W_up, b_up, W_upc, b_upc, W_freq, b_freq, W_amp, b_amp, W_noise, b_noise):
    raise NotImplementedError("write your pallas kernel here")


# speedup vs baseline: 1.4330x; 1.4330x over previous
import jax
import jax.numpy as jnp
from jax.experimental import pallas as pl
from jax.experimental.pallas import tpu as pltpu

MD = 128
NF = 128
NT = 128
NA = 16
AS = 8192
NS = 32768
B = 2
NYQ = 11025.0
DP = jax.lax.Precision.DEFAULT

_f32 = jnp.float32


def _conv2d(x, w, b, stride=1, padding=1):
    out = jax.lax.conv_general_dilated(
        x, w, (stride, stride), [(padding, padding), (padding, padding)],
        dimension_numbers=('NCHW', 'OIHW', 'NCHW'))
    return out + b[None, :, None, None]


def _up2(x):
    return jnp.repeat(jnp.repeat(x, 2, axis=2), 2, axis=3)


def _values_body(xx_ref, wval_ref, bval_ref, val_ref):
    RB = 16

    def tile(i, carry):
        xx2 = xx_ref[0, pl.ds(i * RB, RB), :, :].reshape(RB * NT, MD)
        val_ref[0, pl.ds(i * RB * NT, RB * NT), :] = (
            jnp.dot(xx2, wval_ref[...], preferred_element_type=_f32,
                    precision=DP) + bval_ref[0][None, :])
        return carry

    jax.lax.fori_loop(0, NF // RB, tile, 0)


def _topk_body(s_ref, val_ref, sm_ref, lat_ref, idx_ref):
    x = s_ref[0]
    m = jnp.max(x)
    e = jnp.exp(x - m)
    sm_ref[0] = e / jnp.sum(e)
    ii = jax.lax.broadcasted_iota(jnp.int32, (NF, NT), 0)
    jj = jax.lax.broadcasted_iota(jnp.int32, (NF, NT), 1)
    lin = ii * NT + jj
    cur = x
    neg = _f32(-jnp.inf)
    idxs = []
    rows = []
    for _ in range(NA):
        v = jnp.max(cur)
        j = jnp.min(jnp.where(cur == v, lin, jnp.int32(1 << 30)))
        idxs.append(j)
        cur = jnp.where(lin == j, neg, cur)
        rows.append(val_ref[0, pl.ds(j, 1), :])
    lat_ref[0] = jnp.concatenate(rows, axis=0)
    idx_ref[...] = jnp.stack(idxs)[None, None, :]


def _audio_body(lat_ref, wup_ref, bup_ref, wc_ref, bupc_ref,
                wf_ref, bf_ref, wa_ref, ba_ref, wn_ref, bn_ref,
                noise_ref, o_ref):
    lat = lat_ref[0]
    hs = [jnp.dot(lat, wup_ref[t], preferred_element_type=_f32, precision=DP)
          + bup_ref[pl.ds(t, 1), :] for t in range(4)]
    rows = []
    for t in range(4):
        rows.extend([hs[t]] * 8)
    Ht = jnp.concatenate(rows, axis=0)
    z = jnp.zeros((1, MD), _f32)
    Hm = jnp.concatenate([z, Ht[:-1]], axis=0)
    Hp = jnp.concatenate([Ht[1:], z], axis=0)
    h2 = (jnp.dot(Hm, wc_ref[0], preferred_element_type=_f32, precision=DP)
          + jnp.dot(Ht, wc_ref[1], preferred_element_type=_f32, precision=DP)
          + jnp.dot(Hp, wc_ref[2], preferred_element_type=_f32, precision=DP)
          + bupc_ref[...])
    zf = jnp.dot(h2, wf_ref[...], preferred_element_type=_f32,
                 precision=DP) + bf_ref[...]
    sig = 1.0 / (1.0 + jnp.exp(-zf))
    lowest = _f32(30.0 / NYQ)
    f = lowest + sig * (1.0 - lowest)
    a = (jnp.dot(h2, wa_ref[...], preferred_element_type=_f32, precision=DP)
         + ba_ref[...]) ** 2
    nz = (jnp.dot(h2, wn_ref[...], preferred_element_type=_f32, precision=DP)
          + bn_ref[...]) ** 2
    env = jnp.mean(nz, axis=1, keepdims=True)

    fp = f * _f32(jnp.pi)
    c = fp
    for sh in (1, 2, 4, 8, 16):
        c = c + jnp.concatenate([jnp.zeros((sh, MD), _f32), c[:-sh]], axis=0)
    cex = c - fp
    fpT = fp.T
    cexT = cex.T
    kramp = (jax.lax.broadcasted_iota(jnp.int32, (1, 1, 256), 2) + 1
             ).astype(_f32)
    P = _f32(256.0) * cexT[:, :, None] + fpT[:, :, None] * kramp
    S = jnp.sin(P) * a.T[:, :, None]
    harm = jnp.sum(S, axis=0)
    o_ref[0] = harm + env * noise_ref[0]


def _scatter_body(atoms_ref, idx_ref, o_ref):
    b = pl.program_id(0)
    o_ref[0] = jnp.zeros((320, MD), _f32)
    lane = jax.lax.broadcasted_iota(jnp.int32, (65, MD), 1)
    zr = jnp.zeros((1, MD), _f32)
    for a in range(NA):
        j = idx_ref[b, 0, a]
        q = j // MD
        r = j % MD
        A = atoms_ref[0, a]
        aug1 = jnp.concatenate([A, zr], axis=0)
        aug2 = jnp.concatenate([zr, A], axis=0)
        b1 = pltpu.roll(aug1, r, 1)
        b2 = pltpu.roll(aug2, r, 1)
        S = jnp.where(lane >= r, b1, b2)
        o_ref[0, pl.ds(q, 65), :] += S


def kernel(x, pos, W_initial, b_initial, W_withpos, b_withpos, W_s1, b_s1,
           W_s2, b_s2, W_s3, b_s3, W_s4, b_s4, W_sbconv, b_sbconv,
           W_values, b_values, W_select, b_select, W_up, b_up, W_upc, b_upc,
           W_freq, b_freq, W_amp, b_amp, W_noise, b_noise):
    f32 = _f32

    h = _conv2d(x, W_initial, b_initial)
    p = jnp.broadcast_to(pos[None], (B,) + pos.shape)
    h = jnp.concatenate([h, p], axis=1)
    h = _conv2d(h, W_withpos, b_withpos)
    h = jax.nn.leaky_relu(_conv2d(h, W_s1, b_s1, stride=2), 0.2)
    h = jax.nn.leaky_relu(_conv2d(h, W_s2, b_s2, stride=2), 0.2)
    h = _up2(h)
    h = jax.nn.leaky_relu(_conv2d(h, W_s3, b_s3), 0.2)
    h = _up2(h)
    h = jax.nn.leaky_relu(_conv2d(h, W_s4, b_s4), 0.2)
    xx = _conv2d(h, W_sbconv, b_sbconv)
    norms = jnp.linalg.norm(xx, axis=1, keepdims=True)
    normed = xx / (norms + 1e-08)
    s = _conv2d(normed, W_select, b_select, padding=0)
    s_raw = s.reshape(B, NF, NT)
    xx_hwc = jnp.transpose(xx, (0, 2, 3, 1))

    full = lambda s_: pl.BlockSpec(s_, lambda b: (0,) * len(s_))
    bmap = lambda s_: pl.BlockSpec((1,) + s_, lambda b: (b,) + (0,) * len(s_))

    wval = W_values.reshape(MD, MD).T
    values = pl.pallas_call(
        _values_body,
        grid=(B,),
        in_specs=[bmap((NF, NT, MD)), full((MD, MD)), full((1, MD))],
        out_specs=bmap((NF * NT, MD)),
        out_shape=jax.ShapeDtypeStruct((B, NF * NT, MD), f32),
    )(xx_hwc, wval, b_values.reshape(1, MD))

    feat_map, latents, idx = pl.pallas_call(
        _topk_body,
        grid=(B,),
        in_specs=[bmap((NF, NT)), bmap((NF * NT, MD))],
        out_specs=(bmap((NF, NT)), bmap((NA, MD)), bmap((1, NA))),
        out_shape=(jax.ShapeDtypeStruct((B, NF, NT), f32),
                   jax.ShapeDtypeStruct((B, NA, MD), f32),
                   jax.ShapeDtypeStruct((B, 1, NA), jnp.int32)),
    )(s_raw, values)

    lat_flat = latents.reshape(B * NA, 1, MD)
    wup = jnp.transpose(W_up.reshape(MD, MD, 4), (2, 0, 1))
    bup = b_up.reshape(MD, 4).T
    wc = jnp.transpose(W_upc, (2, 1, 0))
    noise = jax.random.uniform(jax.random.key(42), (B * NA, AS),
                               minval=-1.0, maxval=1.0,
                               dtype=f32).reshape(B * NA, 32, 256)

    atoms = pl.pallas_call(
        _audio_body,
        grid=(B * NA,),
        in_specs=[bmap((1, MD)), full((4, MD, MD)), full((4, MD)),
                  full((3, MD, MD)), full((1, MD)),
                  full((MD, MD)), full((1, MD)),
                  full((MD, MD)), full((1, MD)),
                  full((MD, MD)), full((1, MD)),
                  bmap((32, 256))],
        out_specs=bmap((32, 256)),
        out_shape=jax.ShapeDtypeStruct((B * NA, 32, 256), f32),
    )(lat_flat, wup, bup, wc, b_upc.reshape(1, MD),
      W_freq.T, b_freq.reshape(1, MD), W_amp.T, b_amp.reshape(1, MD),
      W_noise.T, b_noise.reshape(1, MD), noise)

    acc = pl.pallas_call(
        _scatter_body,
        grid=(B,),
        in_specs=[bmap((NA, AS // MD, MD)),
                  pl.BlockSpec(memory_space=pltpu.SMEM)],
        out_specs=bmap((320, MD)),
        out_shape=jax.ShapeDtypeStruct((B, 320, MD), f32),
    )(atoms.reshape(B, NA, AS // MD, MD), idx)

    out = acc.reshape(B, 320 * MD)[:, :NS][:, None, :]
    return out, latents, feat_map
